# baseline (speedup 1.0000x reference)
import jax
import jax.numpy as jnp
from jax.experimental import pallas as pl
from jax.experimental.pallas import tpu as pltpu

_B, _N, _I, _O = 16, 1024, 256, 256


def _fused_kernel(adj_ref, inv_deg_ref, x_ref, m_ref, ocols_ref, weight_ref,
                  bias_ref, w_ih_t_ref, b_ref, out_ref):
    n, i_dim, o_dim = _N, _I, _O
    adj = adj_ref[0]
    x = x_ref[0]
    m = m_ref[...]

    w_adj = (adj * m).astype(jnp.bfloat16)
    sum_x = jax.lax.dot_general(
        w_adj, x.astype(jnp.bfloat16), (((0,), (0,)), ((), ())),
        preferred_element_type=jnp.float32)
    mean_x = jax.lax.dot_general(
        inv_deg_ref[0].astype(jnp.bfloat16), sum_x.astype(jnp.bfloat16),
        (((1,), (0,)), ((), ())),
        preferred_element_type=jnp.float32)

    cols = adj[:, n - 2:n]
    ocols = ocols_ref[...]
    neigh = cols == 1.0
    vals = jnp.where(neigh, ocols, jnp.inf)
    minv = jnp.min(vals, axis=0, keepdims=True)
    iota = jax.lax.broadcasted_iota(jnp.int32, (n, 2), 0)
    cand = neigh & (vals == minv)
    last_idx = jnp.max(jnp.where(cand, iota, -1), axis=0, keepdims=True)
    has = jnp.max(jnp.where(neigh, 1.0, 0.0), axis=0, keepdims=True)
    onehot = (iota == last_idx).astype(jnp.float32)
    selx = jax.lax.dot_general(
        onehot, x, (((0,), (0,)), ((), ())),
        preferred_element_type=jnp.float32)
    gates = jnp.dot(selx, w_ih_t_ref[...],
                    preferred_element_type=jnp.float32) + b_ref[...]
    gi = gates[:, 0:i_dim]
    gg = gates[:, 2 * i_dim:3 * i_dim]
    go = gates[:, 3 * i_dim:4 * i_dim]
    c = jax.nn.sigmoid(gi) * jnp.tanh(gg)
    h = jax.nn.sigmoid(go) * jnp.tanh(c)
    h = h * has.T

    row = jax.lax.broadcasted_iota(jnp.int32, (n, i_dim), 0)
    merged = jnp.where(row < n - 2, mean_x, 0.0)
    merged = jnp.where(row == n - 2, h[0:1, :], merged)
    merged = jnp.where(row == n - 1, h[1:2, :], merged)

    new_x = jnp.dot(merged, weight_ref[...],
                    preferred_element_type=jnp.float32)
    self_x = jnp.dot(x, bias_ref[...],
                     preferred_element_type=jnp.float32)
    ss = (jnp.sum(new_x * new_x, axis=1, keepdims=True)
          + jnp.sum(self_x * self_x, axis=1, keepdims=True))
    inv = 1.0 / jnp.maximum(jnp.sqrt(ss), 1e-12)
    out_ref[0, :, 0:o_dim] = new_x * inv
    out_ref[0, :, o_dim:2 * o_dim] = self_x * inv


def kernel(x, adj, inv_deg, weight, bias, weight_adj, order_adj, w_ih, w_hh,
           b_ih, b_hh):
    n = _N
    m = weight_adj.at[:, n - 2:].set(1.0).at[n - 2:, :].set(1.0)
    ocols = order_adj[:, n - 2:]
    w_ih_t = w_ih.T
    b = (b_ih + b_hh)[None, :]

    grid = (_B,)
    out = pl.pallas_call(
        _fused_kernel,
        grid=grid,
        in_specs=[
            pl.BlockSpec((1, n, n), lambda b_: (b_, 0, 0)),
            pl.BlockSpec((1, n, n), lambda b_: (b_, 0, 0)),
            pl.BlockSpec((1, n, _I), lambda b_: (b_, 0, 0)),
            pl.BlockSpec((n, n), lambda b_: (0, 0)),
            pl.BlockSpec((n, 2), lambda b_: (0, 0)),
            pl.BlockSpec((_I, _O), lambda b_: (0, 0)),
            pl.BlockSpec((_I, _O), lambda b_: (0, 0)),
            pl.BlockSpec((_I, 4 * _I), lambda b_: (0, 0)),
            pl.BlockSpec((1, 4 * _I), lambda b_: (0, 0)),
        ],
        out_specs=pl.BlockSpec((1, n, 2 * _O), lambda b_: (b_, 0, 0)),
        out_shape=jax.ShapeDtypeStruct((_B, n, 2 * _O), jnp.float32),
    )(adj, inv_deg, x, m, ocols, weight, bias, w_ih_t, b)
    return out


# speedup vs baseline: 1.1480x; 1.1480x over previous
import jax
import jax.numpy as jnp
from jax.experimental import pallas as pl
from jax.experimental.pallas import tpu as pltpu

_B, _N, _I, _O = 16, 1024, 256, 256


def _fused_kernel(adj_ref, inv_deg_ref, x_ref, wa_ref, oadj_ref, weight_ref,
                  bias_ref, w_ih_ref, b_ih_ref, b_hh_ref, out_ref, m_s):
    n, i_dim, o_dim = _N, _I, _O

    @pl.when(pl.program_id(0) == 0)
    def _init():
        r = jax.lax.broadcasted_iota(jnp.int32, (n, n), 0)
        c = jax.lax.broadcasted_iota(jnp.int32, (n, n), 1)
        band = (r >= n - 2) | (c >= n - 2)
        m_s[...] = jnp.where(band, 1.0, wa_ref[...])

    adj = adj_ref[0]
    x = x_ref[0]

    w_adj = adj * m_s[...]
    sum_x = jax.lax.dot_general(
        w_adj, x, (((0,), (0,)), ((), ())),
        preferred_element_type=jnp.float32)
    mean_x = jax.lax.dot_general(
        inv_deg_ref[0], sum_x, (((1,), (0,)), ((), ())),
        preferred_element_type=jnp.float32)

    cols = adj[:, n - 2:n]
    ocols = oadj_ref[:, n - 2:n]
    neigh = cols == 1.0
    vals = jnp.where(neigh, ocols, jnp.inf)
    minv = jnp.min(vals, axis=0, keepdims=True)
    iota = jax.lax.broadcasted_iota(jnp.int32, (n, 2), 0)
    cand = neigh & (vals == minv)
    last_idx = jnp.max(jnp.where(cand, iota, -1), axis=0, keepdims=True)
    has = jnp.max(jnp.where(neigh, 1.0, 0.0), axis=0, keepdims=True)
    onehot = (iota == last_idx).astype(jnp.float32)
    selx = jax.lax.dot_general(
        onehot, x, (((0,), (0,)), ((), ())),
        preferred_element_type=jnp.float32)
    gates = jax.lax.dot_general(
        selx, w_ih_ref[...], (((1,), (1,)), ((), ())),
        preferred_element_type=jnp.float32)
    gates = gates + b_ih_ref[...] + b_hh_ref[...]
    gi = gates[:, 0:i_dim]
    gg = gates[:, 2 * i_dim:3 * i_dim]
    go = gates[:, 3 * i_dim:4 * i_dim]
    c = jax.nn.sigmoid(gi) * jnp.tanh(gg)
    h = jax.nn.sigmoid(go) * jnp.tanh(c)
    h = h * has.T

    row = jax.lax.broadcasted_iota(jnp.int32, (n, i_dim), 0)
    merged = jnp.where(row < n - 2, mean_x, 0.0)
    merged = jnp.where(row == n - 2, h[0:1, :], merged)
    merged = jnp.where(row == n - 1, h[1:2, :], merged)

    new_x = jnp.dot(merged, weight_ref[...],
                    preferred_element_type=jnp.float32)
    self_x = jnp.dot(x, bias_ref[...],
                     preferred_element_type=jnp.float32)
    ss = (jnp.sum(new_x * new_x, axis=1, keepdims=True)
          + jnp.sum(self_x * self_x, axis=1, keepdims=True))
    inv = 1.0 / jnp.maximum(jnp.sqrt(ss), 1e-12)
    out_ref[0, :, 0:o_dim] = new_x * inv
    out_ref[0, :, o_dim:2 * o_dim] = self_x * inv


def kernel(x, adj, inv_deg, weight, bias, weight_adj, order_adj, w_ih, w_hh,
           b_ih, b_hh):
    n = _N
    b_ih2 = b_ih.reshape(1, 4 * _I)
    b_hh2 = b_hh.reshape(1, 4 * _I)

    grid = (_B,)
    out = pl.pallas_call(
        _fused_kernel,
        grid=grid,
        in_specs=[
            pl.BlockSpec((1, n, n), lambda b_: (b_, 0, 0)),
            pl.BlockSpec((1, n, n), lambda b_: (b_, 0, 0)),
            pl.BlockSpec((1, n, _I), lambda b_: (b_, 0, 0)),
            pl.BlockSpec((n, n), lambda b_: (0, 0)),
            pl.BlockSpec((n, n), lambda b_: (0, 0)),
            pl.BlockSpec((_I, _O), lambda b_: (0, 0)),
            pl.BlockSpec((_I, _O), lambda b_: (0, 0)),
            pl.BlockSpec((4 * _I, _I), lambda b_: (0, 0)),
            pl.BlockSpec((1, 4 * _I), lambda b_: (0, 0)),
            pl.BlockSpec((1, 4 * _I), lambda b_: (0, 0)),
        ],
        out_specs=pl.BlockSpec((1, n, 2 * _O), lambda b_: (b_, 0, 0)),
        out_shape=jax.ShapeDtypeStruct((_B, n, 2 * _O), jnp.float32),
        scratch_shapes=[pltpu.VMEM((n, n), jnp.float32)],
    )(adj, inv_deg, x, weight_adj, order_adj, weight, bias, w_ih, b_ih2, b_hh2)
    return out
